# baseline (speedup 1.0000x reference)
import functools

import jax
import jax.numpy as jnp
from jax import lax
from jax.experimental import pallas as pl
from jax.experimental.pallas import tpu as pltpu
from jax.experimental.pallas import tpu_sc as plsc

BATCH = 16384
DIM = 64
MARGIN = 1.0
PAIR = 128

NW = 32
NROWS = 1000000

CHUNK = 256
NCH = NROWS // CHUNK
TAILE = NROWS - NCH * CHUNK
CPW = 122
TSLOTS = 124
SRING = 8

N_ENT = 4 * BATCH
N_REL = 2 * BATCH
N_ALL = N_ENT + N_REL
DUMP = N_ALL
OUT_ROWS = 100352

LBLK = 512


def _iota16():
    return lax.iota(jnp.int32, 16)


def _lane(vec, j):
    return jnp.sum(jnp.where(_iota16() == j, vec, 0))


def _sc_extract_body(entT, relT, sie, sde, sir, sdr, wse, wsr, out,
                     slab, tslab, lbi, lbd, wsv, staging, dst2d,
                     sl0, sl1, sl2, sl3,
                     sc0, sc1, sc2, sc3, sc4, sc5, sc6, sc7):
    wid = lax.axis_index("s") * 2 + lax.axis_index("c")
    slab_sems = (sl0, sl1, sl2, sl3)
    sc_sems = (sc0, sc1, sc2, sc3, sc4, sc5, sc6, sc7)

    def ws_at(ws_ref, k):
        pltpu.sync_copy(ws_ref.at[pl.ds((k // 16) * 16, 16)], wsv)
        return _lane(wsv[...], k % 16)

    def window_passes(slab_ref, width, sidx, sdst, s0, s1, e0, live0, state):
        hi = e0 + width

        def cond(st):
            return st[3]

        def body(st):
            wp, blk, nscat, _ = st
            wblk = (16 * wp) // LBLK

            @pl.when(wblk != blk)
            def _():
                pltpu.sync_copy(sidx.at[pl.ds(wblk * LBLK, LBLK)], lbi)
                pltpu.sync_copy(sdst.at[pl.ds(wblk * LBLK, LBLK)], lbd)

            ow = (16 * wp) - wblk * LBLK
            idxw = lbi[pl.ds(ow, 16)]
            dstw = lbd[pl.ds(ow, 16)]
            posv = _iota16() + 16 * wp
            valid = ((posv >= s0) & (posv < s1)
                     & (idxw >= e0) & (idxw < hi))
            anyv = jnp.max(jnp.where(valid, 1, 0)) == 1
            cols = jnp.clip(idxw - e0, 0, width - 1)
            r = nscat % SRING

            @pl.when(anyv)
            def _():
                for rr in range(SRING):
                    @pl.when(r == rr)
                    def _(rr=rr):
                        @pl.when(nscat >= SRING)
                        def _():
                            pltpu.make_async_copy(
                                staging.at[rr], out.at[dst2d.at[rr]],
                                sc_sems[rr]).wait()
                rv = jnp.full((16,), r, jnp.int32)
                for d in range(DIM):
                    v = plsc.load_gather(
                        slab_ref, [jnp.full((16,), d, jnp.int32), cols])
                    plsc.store_scatter(
                        staging,
                        [rv, _iota16(), jnp.full((16,), d, jnp.int32)], v)
                dst2d[r, pl.ds(0, 16)] = jnp.where(valid, dstw, DUMP)
                for rr in range(SRING):
                    @pl.when(r == rr)
                    def _(rr=rr):
                        pltpu.make_async_copy(
                            staging.at[rr], out.at[dst2d.at[rr]],
                            sc_sems[rr]).start()

            nscat2 = jnp.where(anyv, nscat + 1, nscat)
            bad = jnp.max(jnp.where((idxw >= hi) & (posv < s1), 1, 0))
            done = bad == 0
            wp2 = jnp.where(done, wp + 1, wp)
            more = done & (16 * wp2 < s1)
            return (wp2, wblk, nscat2, more)

        wp, blk, nscat, _ = lax.while_loop(
            cond, body, (state[0], state[1], state[2],
                         live0 & (16 * state[0] < s1)))
        return (wp, blk, nscat)

    def phase(tab, sidx, sdst, ws_ref, nscat_in):
        s0 = ws_at(ws_ref, wid)
        s1 = ws_at(ws_ref, wid + 1)
        c_lo = wid * CPW
        c_hi = jnp.minimum(c_lo + CPW + 2, NCH)

        def slab_descs(c, b):
            return [pltpu.make_async_copy(
                tab.at[pl.ds(8 * g, 8), pl.ds(c * CHUNK, CHUNK)],
                slab.at[b, pl.ds(8 * g, 8)], slab_sems[b])
                for g in range(DIM // 8)]

        def slab_start(c, b):
            for d_ in slab_descs(c, b):
                d_.start()

        def slab_wait(c, b):
            for d_ in slab_descs(c, b):
                d_.wait()

        for p in range(3):
            @pl.when(c_lo + p < c_hi)
            def _(p=p):
                slab_start(c_lo + p, p)

        state0 = (s0 // 16, jnp.int32(-1), nscat_in)

        @pl.loop(0, TSLOTS // 4, init_carry=state0)
        def chunk_loop(tt, carry):
            for b in range(4):
                t = 4 * tt + b
                c = c_lo + t

                @pl.when(c + 3 < c_hi)
                def _():
                    slab_start(c + 3, (b + 3) % 4)

                live = c < c_hi

                @pl.when(live)
                def _():
                    slab_wait(c, b)

                carry = window_passes(
                    slab.at[b], CHUNK, sidx, sdst, s0, s1,
                    c * CHUNK, live, carry)
            return carry

        wp, blk, nscat = chunk_loop
        live_t = wid == NW - 1

        @pl.when(live_t)
        def _():
            pltpu.sync_copy(tab.at[:, pl.ds(NCH * CHUNK, TAILE)], tslab)

        wp, blk, nscat = window_passes(
            tslab, TAILE, sidx, sdst, s0, s1,
            NCH * CHUNK, live_t, (wp, blk, nscat))
        return nscat

    nscat = phase(entT, sie, sde, wse, jnp.int32(0))
    nscat = phase(relT, sir, sdr, wsr, nscat)

    for d in range(1, SRING + 1):
        k = nscat - d
        r = k % SRING
        for rr in range(SRING):
            @pl.when((k >= 0) & (r == rr))
            def _(rr=rr):
                pltpu.make_async_copy(
                    staging.at[rr], out.at[dst2d.at[rr]],
                    sc_sems[rr]).wait()


def _make_sc_extract():
    mesh = plsc.VectorSubcoreMesh(core_axis_name="c", subcore_axis_name="s")
    return functools.partial(
        pl.kernel, mesh=mesh,
        compiler_params=pltpu.CompilerParams(needs_layout_passes=False),
        out_type=jax.ShapeDtypeStruct((OUT_ROWS, PAIR), jnp.float32),
        scratch_types=[
            pltpu.VMEM((4, DIM, CHUNK), jnp.float32),
            pltpu.VMEM((DIM, TAILE), jnp.float32),
            pltpu.VMEM((LBLK,), jnp.int32),
            pltpu.VMEM((LBLK,), jnp.int32),
            pltpu.VMEM((16,), jnp.int32),
            pltpu.VMEM((SRING, 16, PAIR), jnp.float32),
            pltpu.VMEM((SRING, 16), jnp.int32),
        ] + [pltpu.SemaphoreType.DMA] * (4 + SRING),
    )(_sc_extract_body)


_sc_extract = _make_sc_extract()

CB = 2048
NBLK = BATCH // CB


def _score_body(ph, pt, pr, nh, nt, nr, out):
    k = pl.program_id(0)

    def score(h_ref, t_ref, r_ref):
        h = h_ref[...][:, :DIM]
        t = t_ref[...][:, :DIM]
        r = r_ref[...][:, :DIM]
        hn = h / jnp.sqrt(jnp.sum(h * h, axis=1, keepdims=True))
        tn = t / jnp.sqrt(jnp.sum(t * t, axis=1, keepdims=True))
        diff = hn + r - tn
        return jnp.sqrt(jnp.sum(diff * diff, axis=1))

    ps = score(ph, pt, pr)
    ns = score(nh, nt, nr)
    part = jnp.sum(jnp.maximum(MARGIN + ps - ns, 0.0)).reshape(1, 1)
    prev = jnp.where(k == 0, jnp.zeros((1, 1), jnp.float32), out[...])
    total = prev + part
    out[...] = jnp.where(k == NBLK - 1, total / BATCH, total)


def _tc_score(rows):
    def blk(off):
        return pl.BlockSpec((CB, PAIR), lambda k, o=off: (k + o, 0))

    out = pl.pallas_call(
        _score_body,
        grid=(NBLK,),
        in_specs=[blk(0), blk(NBLK), blk(4 * NBLK), blk(2 * NBLK),
                  blk(3 * NBLK), blk(5 * NBLK)],
        out_specs=pl.BlockSpec((1, 1), lambda k: (0, 0)),
        out_shape=jax.ShapeDtypeStruct((1, 1), jnp.float32),
    )(rows, rows, rows, rows, rows, rows)
    return out.reshape(())


def kernel(pos_x, neg_x, ent_table, rel_table):
    ent_idx = jnp.concatenate(
        [pos_x[:, 0], pos_x[:, 1], neg_x[:, 0], neg_x[:, 1]])
    rel_idx = jnp.concatenate([pos_x[:, 2], neg_x[:, 2]])

    eperm = jnp.argsort(ent_idx)
    sie = ent_idx[eperm]
    sde = eperm.astype(jnp.int32)
    rperm = jnp.argsort(rel_idx)
    sir = rel_idx[rperm]
    sdr = rperm.astype(jnp.int32) + N_ENT

    bounds = jnp.concatenate(
        [jnp.arange(32, dtype=jnp.int32) * (CPW * CHUNK),
         jnp.array([NROWS], jnp.int32)])
    wse = jnp.zeros((48,), jnp.int32).at[:33].set(
        jnp.searchsorted(sie, bounds).astype(jnp.int32))
    wsr = jnp.zeros((48,), jnp.int32).at[:33].set(
        jnp.searchsorted(sir, bounds).astype(jnp.int32))

    rows = _sc_extract(ent_table.T, rel_table.T, sie, sde, sir, sdr, wse, wsr)
    return _tc_score(rows)


# speedup vs baseline: 1.6742x; 1.6742x over previous
import functools

import jax
import jax.numpy as jnp
from jax import lax
from jax.experimental import pallas as pl
from jax.experimental.pallas import tpu as pltpu
from jax.experimental.pallas import tpu_sc as plsc

BATCH = 16384
DIM = 64
MARGIN = 1.0
PAIR = 128

NW = 32
NROWS = 1000000

CHUNK = 512
NCH = NROWS // CHUNK
TAILE = NROWS - NCH * CHUNK
CPW = 61
TSLOTS = 64

BWIN = 8

N_ENT = 4 * BATCH
N_REL = 2 * BATCH
N_ALL = N_ENT + N_REL
DUMP = N_ALL
OUT_ROWS = 100352

LBLK = 512


def _iota16():
    return lax.iota(jnp.int32, 16)


def _lane(vec, j):
    return jnp.sum(jnp.where(_iota16() == j, vec, 0))


def _sc_extract_body(entT, relT, sie, sde, sir, sdr, wse, wsr, out,
                     slab, tslab, lbi, lbd, wsv, staging, dst2,
                     sl0, sl1, sc0, sc1):
    wid = lax.axis_index("s") * 2 + lax.axis_index("c")
    slab_sems = (sl0, sl1)
    sc_sems = (sc0, sc1)

    def ws_at(ws_ref, k):
        pltpu.sync_copy(ws_ref.at[pl.ds((k // 16) * 16, 16)], wsv)
        return _lane(wsv[...], k % 16)

    def sc_wait(ss):
        pltpu.make_async_copy(
            staging.at[ss], out.at[dst2.at[ss]], sc_sems[ss]).wait()

    def sc_start(ss):
        pltpu.make_async_copy(
            staging.at[ss], out.at[dst2.at[ss]], sc_sems[ss]).start()

    def window_passes(slab_ref, width, sidx, sdst, s0, s1, e0, live0, state):
        hi = e0 + width

        def cond(st):
            return st[4]

        def body(st):
            wp, blk, nscat, f, _ = st
            wblk = (16 * wp) // LBLK

            @pl.when(wblk != blk)
            def _():
                pltpu.sync_copy(sidx.at[pl.ds(wblk * LBLK, LBLK)], lbi)
                pltpu.sync_copy(sdst.at[pl.ds(wblk * LBLK, LBLK)], lbd)

            ow = (16 * wp) - wblk * LBLK
            idxw = lbi[pl.ds(ow, 16)]
            dstw = lbd[pl.ds(ow, 16)]
            posv = _iota16() + 16 * wp
            valid = ((posv >= s0) & (posv < s1)
                     & (idxw >= e0) & (idxw < hi))
            anyv = jnp.max(jnp.where(valid, 1, 0)) == 1
            cols = jnp.clip(idxw - e0, 0, width - 1)
            slot = nscat % 2

            @pl.when(anyv)
            def _():
                @pl.when((f == 0) & (nscat >= 2))
                def _():
                    for ss in range(2):
                        @pl.when(slot == ss)
                        def _(ss=ss):
                            sc_wait(ss)

                slot_v = jnp.full((16,), slot, jnp.int32)
                row_v = 16 * f + _iota16()
                for d in range(DIM):
                    v = plsc.load_gather(
                        slab_ref, [jnp.full((16,), d, jnp.int32), cols])
                    plsc.store_scatter(
                        staging,
                        [slot_v, row_v, jnp.full((16,), d, jnp.int32)], v)
                plsc.store_scatter(
                    dst2, [slot_v, row_v], jnp.where(valid, dstw, DUMP))

                @pl.when(f == BWIN - 1)
                def _():
                    for ss in range(2):
                        @pl.when(slot == ss)
                        def _(ss=ss):
                            sc_start(ss)

            f2 = jnp.where(anyv, f + 1, f)
            full = f2 == BWIN
            nscat2 = jnp.where(full, nscat + 1, nscat)
            f3 = jnp.where(full, 0, f2)

            bad = jnp.max(jnp.where((idxw >= hi) & (posv < s1), 1, 0))
            done = bad == 0
            wp2 = jnp.where(done, wp + 1, wp)
            more = done & (16 * wp2 < s1)
            return (wp2, wblk, nscat2, f3, more)

        wp, blk, nscat, f, _ = lax.while_loop(
            cond, body, (state[0], state[1], state[2], state[3],
                         live0 & (16 * state[0] < s1)))
        return (wp, blk, nscat, f)

    def phase(tab, sidx, sdst, ws_ref, state_in):
        s0 = ws_at(ws_ref, wid)
        s1 = ws_at(ws_ref, wid + 1)
        c_lo = wid * CPW
        c_hi = jnp.minimum(c_lo + CPW + 2, NCH)

        def slab_descs(c, b):
            return [pltpu.make_async_copy(
                tab.at[pl.ds(8 * g, 8), pl.ds(c * CHUNK, CHUNK)],
                slab.at[b, pl.ds(8 * g, 8)], slab_sems[b])
                for g in range(DIM // 8)]

        def slab_start(c, b):
            for d_ in slab_descs(c, b):
                d_.start()

        def slab_wait(c, b):
            for d_ in slab_descs(c, b):
                d_.wait()

        slab_start(c_lo, 0)

        state0 = (s0 // 16, jnp.int32(-1), state_in[2], state_in[3])

        @pl.loop(0, TSLOTS // 2, init_carry=state0)
        def chunk_loop(tt, carry):
            for b in range(2):
                t = 2 * tt + b
                c = c_lo + t

                @pl.when(c + 1 < c_hi)
                def _():
                    slab_start(c + 1, 1 - b)

                live = c < c_hi

                @pl.when(live)
                def _():
                    slab_wait(c, b)

                carry = window_passes(
                    slab.at[b], CHUNK, sidx, sdst, s0, s1,
                    c * CHUNK, live, carry)
            return carry

        state1 = chunk_loop
        live_t = wid == NW - 1

        @pl.when(live_t)
        def _():
            pltpu.sync_copy(tab.at[:, pl.ds(NCH * CHUNK, TAILE)], tslab)

        return window_passes(
            tslab, TAILE, sidx, sdst, s0, s1,
            NCH * CHUNK, live_t, state1)

    st = (jnp.int32(0), jnp.int32(-1), jnp.int32(0), jnp.int32(0))
    st = phase(entT, sie, sde, wse, st)
    st = phase(relT, sir, sdr, wsr, st)
    _, _, nscat, f = st

    @pl.when(f > 0)
    def _():
        for ss in range(2):
            @pl.when((nscat % 2) == ss)
            def _(ss=ss):
                for seg in range(BWIN):
                    @pl.when(seg >= f)
                    def _(seg=seg):
                        dst2[ss, pl.ds(16 * seg, 16)] = jnp.full(
                            (16,), DUMP, jnp.int32)
                sc_start(ss)

    nfin = jnp.where(f > 0, nscat + 1, nscat)

    for d in range(1, 3):
        k = nfin - d
        for ss in range(2):
            @pl.when((k >= 0) & ((k % 2) == ss))
            def _(ss=ss):
                sc_wait(ss)


def _make_sc_extract():
    mesh = plsc.VectorSubcoreMesh(core_axis_name="c", subcore_axis_name="s")
    return functools.partial(
        pl.kernel, mesh=mesh,
        compiler_params=pltpu.CompilerParams(needs_layout_passes=False),
        out_type=jax.ShapeDtypeStruct((OUT_ROWS, PAIR), jnp.float32),
        scratch_types=[
            pltpu.VMEM((2, DIM, CHUNK), jnp.float32),
            pltpu.VMEM((DIM, TAILE), jnp.float32),
            pltpu.VMEM((LBLK,), jnp.int32),
            pltpu.VMEM((LBLK,), jnp.int32),
            pltpu.VMEM((16,), jnp.int32),
            pltpu.VMEM((2, 16 * BWIN, PAIR), jnp.float32),
            pltpu.VMEM((2, 16 * BWIN), jnp.int32),
        ] + [pltpu.SemaphoreType.DMA] * 4,
    )(_sc_extract_body)


_sc_extract = _make_sc_extract()

CB = 2048
NBLK = BATCH // CB


def _score_body(ph, pt, pr, nh, nt, nr, out):
    k = pl.program_id(0)

    def score(h_ref, t_ref, r_ref):
        h = h_ref[...][:, :DIM]
        t = t_ref[...][:, :DIM]
        r = r_ref[...][:, :DIM]
        hn = h / jnp.sqrt(jnp.sum(h * h, axis=1, keepdims=True))
        tn = t / jnp.sqrt(jnp.sum(t * t, axis=1, keepdims=True))
        diff = hn + r - tn
        return jnp.sqrt(jnp.sum(diff * diff, axis=1))

    ps = score(ph, pt, pr)
    ns = score(nh, nt, nr)
    part = jnp.sum(jnp.maximum(MARGIN + ps - ns, 0.0)).reshape(1, 1)
    prev = jnp.where(k == 0, jnp.zeros((1, 1), jnp.float32), out[...])
    total = prev + part
    out[...] = jnp.where(k == NBLK - 1, total / BATCH, total)


def _tc_score(rows):
    def blk(off):
        return pl.BlockSpec((CB, PAIR), lambda k, o=off: (k + o, 0))

    out = pl.pallas_call(
        _score_body,
        grid=(NBLK,),
        in_specs=[blk(0), blk(NBLK), blk(4 * NBLK), blk(2 * NBLK),
                  blk(3 * NBLK), blk(5 * NBLK)],
        out_specs=pl.BlockSpec((1, 1), lambda k: (0, 0)),
        out_shape=jax.ShapeDtypeStruct((1, 1), jnp.float32),
    )(rows, rows, rows, rows, rows, rows)
    return out.reshape(())


def kernel(pos_x, neg_x, ent_table, rel_table):
    ent_idx = jnp.concatenate(
        [pos_x[:, 0], pos_x[:, 1], neg_x[:, 0], neg_x[:, 1]])
    rel_idx = jnp.concatenate([pos_x[:, 2], neg_x[:, 2]])

    eperm = jnp.argsort(ent_idx)
    sie = ent_idx[eperm]
    sde = eperm.astype(jnp.int32)
    rperm = jnp.argsort(rel_idx)
    sir = rel_idx[rperm]
    sdr = rperm.astype(jnp.int32) + N_ENT

    bounds = jnp.concatenate(
        [jnp.arange(32, dtype=jnp.int32) * (CPW * CHUNK),
         jnp.array([NROWS], jnp.int32)])
    wse = jnp.zeros((48,), jnp.int32).at[:33].set(
        jnp.searchsorted(sie, bounds).astype(jnp.int32))
    wsr = jnp.zeros((48,), jnp.int32).at[:33].set(
        jnp.searchsorted(sir, bounds).astype(jnp.int32))

    rows = _sc_extract(ent_table.T, rel_table.T, sie, sde, sir, sdr, wse, wsr)
    return _tc_score(rows)


# speedup vs baseline: 4.5107x; 2.6943x over previous
import functools

import jax
import jax.numpy as jnp
from jax import lax
from jax.experimental import pallas as pl
from jax.experimental.pallas import tpu as pltpu
from jax.experimental.pallas import tpu_sc as plsc

BATCH = 16384
DIM = 64
MARGIN = 1.0

NW = 32
ROWS_PER_DMA = 128
PAIR = 2 * DIM

ENT_LOOKUPS = 4 * BATCH
REL_LOOKUPS = 2 * BATCH
ENT_PER_W = ENT_LOOKUPS // NW
REL_PER_W = REL_LOOKUPS // NW
ENT_DMAS = ENT_PER_W // ROWS_PER_DMA
REL_DMAS = REL_PER_W // ROWS_PER_DMA


def _sc_gather_body(ent_t, rel_t, eidx, ridx, ent_out, rel_out,
                    eidx_v, ridx_v, rows_v, sem_a, sem_b):
    wid = lax.axis_index("s") * 2 + lax.axis_index("c")
    pltpu.sync_copy(eidx.at[pl.ds(wid * ENT_DMAS, ENT_DMAS)], eidx_v)
    pltpu.sync_copy(ridx.at[pl.ds(wid * REL_DMAS, REL_DMAS)], ridx_v)

    sems = (sem_a, sem_b)

    def run(table, idx_v, out, n_dmas, out_base):
        pend = [None, None]
        pend[0] = pltpu.async_copy(table.at[idx_v.at[0]], rows_v.at[0], sems[0])
        for j in range(n_dmas):
            if j + 1 < n_dmas:
                b = (j + 1) % 2
                pend[b] = pltpu.async_copy(
                    table.at[idx_v.at[j + 1]], rows_v.at[b], sems[b])
            pend[j % 2].wait()
            pltpu.sync_copy(
                rows_v.at[j % 2],
                out.at[pl.ds(out_base + j * ROWS_PER_DMA, ROWS_PER_DMA)])

    run(ent_t, eidx_v, ent_out, ENT_DMAS, wid * ENT_PER_W)
    run(rel_t, ridx_v, rel_out, REL_DMAS, wid * REL_PER_W)


def _make_sc_gather():
    mesh = plsc.VectorSubcoreMesh(core_axis_name="c", subcore_axis_name="s")
    return functools.partial(
        pl.kernel, mesh=mesh,
        out_type=[
            jax.ShapeDtypeStruct((ENT_LOOKUPS, PAIR), jnp.float32),
            jax.ShapeDtypeStruct((REL_LOOKUPS, PAIR), jnp.float32),
        ],
        scratch_types=[
            pltpu.VMEM((ENT_DMAS, ROWS_PER_DMA), jnp.int32),
            pltpu.VMEM((REL_DMAS, ROWS_PER_DMA), jnp.int32),
            pltpu.VMEM((2, ROWS_PER_DMA, PAIR), jnp.float32),
            pltpu.SemaphoreType.DMA,
            pltpu.SemaphoreType.DMA,
        ],
    )(_sc_gather_body)


_sc_gather = _make_sc_gather()

CB = 2048
NBLK = BATCH // CB


def _score_body(ph, pt, pr, nh, nt, nr, par, out):
    k = pl.program_id(0)
    p = par[...]

    def half(pair_ref, col):
        rows = pair_ref[...]
        lo = rows[:, :DIM]
        hi = rows[:, DIM:]
        return jnp.where(p[:, col:col + 1] > 0, hi, lo)

    def score(h, t, r):
        hn = h / jnp.sqrt(jnp.sum(h * h, axis=1, keepdims=True))
        tn = t / jnp.sqrt(jnp.sum(t * t, axis=1, keepdims=True))
        diff = hn + r - tn
        return jnp.sqrt(jnp.sum(diff * diff, axis=1))

    ps = score(half(ph, 0), half(pt, 1), half(pr, 2))
    ns = score(half(nh, 3), half(nt, 4), half(nr, 5))
    part = jnp.sum(jnp.maximum(MARGIN + ps - ns, 0.0)).reshape(1, 1)
    prev = jnp.where(k == 0, jnp.zeros((1, 1), jnp.float32), out[...])
    total = prev + part
    out[...] = jnp.where(k == NBLK - 1, total / BATCH, total)


def _tc_score(ent_rows, rel_rows, parity):
    def blk(off):
        return pl.BlockSpec((CB, PAIR), lambda k, o=off: (k + o, 0))

    out = pl.pallas_call(
        _score_body,
        grid=(NBLK,),
        in_specs=[blk(0), blk(NBLK), blk(0), blk(2 * NBLK), blk(3 * NBLK),
                  blk(NBLK),
                  pl.BlockSpec((CB, 8), lambda k: (k, 0))],
        out_specs=pl.BlockSpec((1, 1), lambda k: (0, 0)),
        out_shape=jax.ShapeDtypeStruct((1, 1), jnp.float32),
    )(ent_rows, ent_rows, rel_rows, ent_rows, ent_rows, rel_rows, parity)
    return out.reshape(())


def kernel(pos_x, neg_x, ent_table, rel_table):
    ent_idx = jnp.concatenate(
        [pos_x[:, 0], pos_x[:, 1], neg_x[:, 0], neg_x[:, 1]])
    rel_idx = jnp.concatenate([pos_x[:, 2], neg_x[:, 2]])
    cols = [pos_x[:, 0], pos_x[:, 1], pos_x[:, 2],
            neg_x[:, 0], neg_x[:, 1], neg_x[:, 2],
            jnp.zeros((BATCH,), jnp.int32), jnp.zeros((BATCH,), jnp.int32)]
    parity = jnp.stack([c & 1 for c in cols], axis=1).astype(jnp.float32)

    ent_pair_idx = (ent_idx >> 1).reshape(ENT_LOOKUPS // ROWS_PER_DMA,
                                          ROWS_PER_DMA)
    rel_pair_idx = (rel_idx >> 1).reshape(REL_LOOKUPS // ROWS_PER_DMA,
                                          ROWS_PER_DMA)
    ent2 = ent_table.reshape(ent_table.shape[0] // 2, PAIR)
    rel2 = rel_table.reshape(rel_table.shape[0] // 2, PAIR)
    ent_rows, rel_rows = _sc_gather(ent2, rel2, ent_pair_idx, rel_pair_idx)
    return _tc_score(ent_rows, rel_rows, parity)
